# baseline (speedup 1.0000x reference)
import functools

import jax
import jax.numpy as jnp
from jax import lax
from jax.experimental import pallas as pl
from jax.experimental.pallas import tpu as pltpu
from jax.experimental.pallas import tpu_sc as plsc

B = 16384
F = 26
V = 64
D = 32
L = 16

CHUNK = 64
CHUNK_W = CHUNK * F
DBLK = 16
TABLE_W = F * V * D
FV = F * V


def _sc_embed(jbase, tables_flat, nw):
    chunks_per_w = B // CHUNK // nw
    mesh = plsc.VectorSubcoreMesh(core_axis_name="c", subcore_axis_name="s")

    @functools.partial(
        pl.kernel,
        mesh=mesh,
        out_type=jax.ShapeDtypeStruct((D * B * F,), jnp.float32),
        scratch_types=[
            pltpu.VMEM((TABLE_W,), jnp.float32),
            pltpu.VMEM((CHUNK_W,), jnp.int32),
            pltpu.VMEM((2, DBLK, CHUNK_W), jnp.float32),
            pltpu.SemaphoreType.DMA,
            pltpu.SemaphoreType.DMA,
            pltpu.SemaphoreType.DMA,
        ],
        compiler_params=pltpu.CompilerParams(needs_layout_passes=False),
    )
    def k(jbase_hbm, tab_hbm, out_hbm, tab_v, idx_v, out_v,
          sem_in, sem_out0, sem_out1):
        wid = lax.axis_index("s") * 2 + lax.axis_index("c")
        pltpu.async_copy(tab_hbm, tab_v, sem_in).wait()
        sems = (sem_out0, sem_out1)

        def drain(p):
            for dd in range(DBLK):
                pltpu.make_async_copy(
                    out_hbm.at[pl.ds(0, CHUNK_W)], out_v.at[p, dd], sems[p]
                ).wait()

        def chunk_body(c, _):
            b0 = (wid * chunks_per_w + c) * CHUNK
            pltpu.async_copy(
                jbase_hbm.at[pl.ds(b0 * F, CHUNK_W)], idx_v, sem_in
            ).wait()
            for p in range(D // DBLK):
                dlo = p * DBLK

                @pl.when(c > 0)
                def _():
                    drain(p)

                @plsc.parallel_loop(0, CHUNK_W, L, unroll=2)
                def body(g):
                    jv = idx_v[pl.ds(g, L)] + dlo * FV
                    for dd in range(DBLK):
                        v = plsc.load_gather(tab_v, [jv])
                        out_v[p, dd, pl.ds(g, L)] = v
                        if dd + 1 < DBLK:
                            jv = jv + FV

                for dd in range(DBLK):
                    pltpu.async_copy(
                        out_v.at[p, dd],
                        out_hbm.at[pl.ds(((dlo + dd) * B + b0) * F, CHUNK_W)],
                        sems[p],
                    )
            return 0

        lax.fori_loop(0, chunks_per_w, chunk_body, 0)
        for p in range(D // DBLK):
            drain(p)

    return k(jbase, tables_flat)


def kernel(inputs, tables):
    jbase = (inputs.astype(jnp.int32)
             + (jnp.arange(F, dtype=jnp.int32) * V)[None, :])
    jbase = jbase.reshape(B * F)
    tables_flat = jnp.transpose(tables, (2, 0, 1)).reshape(TABLE_W)
    out2 = _sc_embed(jbase, tables_flat, 32)
    return out2.reshape(D, B, F)


# speedup vs baseline: 2.4772x; 2.4772x over previous
import functools

import jax
import jax.numpy as jnp
from jax import lax
from jax.experimental import pallas as pl
from jax.experimental.pallas import tpu as pltpu
from jax.experimental.pallas import tpu_sc as plsc

B = 16384
F = 26
V = 64
D = 32
L = 16

CHUNK = 32
CHUNK_W = CHUNK * F
DBLK = 8
TABLE_W = F * V * D
FV = F * V


def _sc_embed(jbase, tables_flat, nw):
    chunks_per_w = B // CHUNK // nw
    mesh = plsc.VectorSubcoreMesh(core_axis_name="c", subcore_axis_name="s")

    @functools.partial(
        pl.kernel,
        mesh=mesh,
        out_type=jax.ShapeDtypeStruct((D * B, F), jnp.float32),
        scratch_types=[
            pltpu.VMEM((TABLE_W,), jnp.float32),
            pltpu.VMEM((CHUNK_W,), jnp.int32),
            pltpu.VMEM((DBLK, CHUNK, F), jnp.float32),
            pltpu.VMEM((DBLK, CHUNK, F), jnp.float32),
            pltpu.SemaphoreType.DMA,
            pltpu.SemaphoreType.DMA,
            pltpu.SemaphoreType.DMA,
        ],
        compiler_params=pltpu.CompilerParams(needs_layout_passes=False),
    )
    def k(jbase_hbm, tab_hbm, out_hbm, tab_v, idx_v, out_v0, out_v1,
          sem_in, sem_out0, sem_out1):
        wid = lax.axis_index("s") * 2 + lax.axis_index("c")
        pltpu.async_copy(tab_hbm, tab_v, sem_in).wait()
        bufs = (out_v0, out_v1)
        sems = (sem_out0, sem_out1)

        def drain(q):
            for dd in range(DBLK):
                pltpu.make_async_copy(
                    out_hbm.at[pl.ds(0, CHUNK), :], bufs[q].at[dd], sems[q]
                ).wait()

        def chunk_body(c, _):
            b0 = (wid * chunks_per_w + c) * CHUNK
            pltpu.async_copy(
                jbase_hbm.at[pl.ds(b0 * F, CHUNK_W)], idx_v, sem_in
            ).wait()
            for p in range(D // DBLK):
                dlo = p * DBLK
                q = p % 2
                buf = bufs[q]
                if p < 2:
                    @pl.when(c > 0)
                    def _():
                        drain(q)
                else:
                    drain(q)

                @plsc.parallel_loop(0, CHUNK, 1, unroll=2)
                def body(b):
                    ja = idx_v[pl.ds(b * F, L)] + dlo * FV
                    jb = idx_v[pl.ds(b * F + (F - L), L)] + dlo * FV
                    for dd in range(DBLK):
                        va = plsc.load_gather(tab_v, [ja])
                        vb = plsc.load_gather(tab_v, [jb])
                        buf[dd, b, pl.ds(0, L)] = va
                        buf[dd, b, pl.ds(F - L, L)] = vb
                        if dd + 1 < DBLK:
                            ja = ja + FV
                            jb = jb + FV

                for dd in range(DBLK):
                    pltpu.async_copy(
                        buf.at[dd],
                        out_hbm.at[pl.ds((dlo + dd) * B + b0, CHUNK), :],
                        sems[q],
                    )
            return 0

        lax.fori_loop(0, chunks_per_w, chunk_body, 0)
        drain(0)
        drain(1)

    return k(jbase, tables_flat)


def kernel(inputs, tables):
    jbase = (inputs.astype(jnp.int32)
             + (jnp.arange(F, dtype=jnp.int32) * V)[None, :])
    jbase = jbase.reshape(B * F)
    tables_flat = jnp.transpose(tables, (2, 0, 1)).reshape(TABLE_W)
    out2 = _sc_embed(jbase, tables_flat, 32)
    return out2.reshape(D, B, F)
